# baseline (speedup 1.0000x reference)
import jax
import jax.numpy as jnp
from jax.experimental import pallas as pl
from jax.experimental.pallas import tpu as pltpu


def _add_body(x_ref, w_ref, o_ref):
    o_ref[...] = x_ref[...] + w_ref[...][None, :, :]


def kernel(x, pos_embed_weight):
    batch, seq, d = x.shape
    s_blk = min(512, seq)
    n_seq = seq // s_blk
    grid = (n_seq, batch)
    out = pl.pallas_call(
        _add_body,
        grid=grid,
        in_specs=[
            pl.BlockSpec((1, s_blk, d), lambda i, j: (j, i, 0)),
            pl.BlockSpec((s_blk, d), lambda i, j: (i, 0)),
        ],
        out_specs=pl.BlockSpec((1, s_blk, d), lambda i, j: (j, i, 0)),
        out_shape=jax.ShapeDtypeStruct((batch, seq, d), x.dtype),
        compiler_params=pltpu.CompilerParams(
            dimension_semantics=("parallel", "arbitrary"),
        ),
    )(x, pos_embed_weight[:seq])
    return out


# speedup vs baseline: 1.1180x; 1.1180x over previous
import jax
import jax.numpy as jnp
from jax.experimental import pallas as pl
from jax.experimental.pallas import tpu as pltpu


def _add_body(x_ref, w_ref, o_ref):
    o_ref[...] = x_ref[...] + w_ref[...][None, :, :]


def kernel(x, pos_embed_weight):
    batch, seq, d = x.shape
    s_blk = min(1024, seq)
    n_seq = seq // s_blk
    grid = (n_seq, batch)
    out = pl.pallas_call(
        _add_body,
        grid=grid,
        in_specs=[
            pl.BlockSpec((1, s_blk, d), lambda i, j: (j, i, 0)),
            pl.BlockSpec((s_blk, d), lambda i, j: (i, 0)),
        ],
        out_specs=pl.BlockSpec((1, s_blk, d), lambda i, j: (j, i, 0)),
        out_shape=jax.ShapeDtypeStruct((batch, seq, d), x.dtype),
        compiler_params=pltpu.CompilerParams(
            dimension_semantics=("parallel", "arbitrary"),
        ),
    )(x, pos_embed_weight[:seq])
    return out


# speedup vs baseline: 1.1695x; 1.0461x over previous
import jax
import jax.numpy as jnp
from jax.experimental import pallas as pl
from jax.experimental.pallas import tpu as pltpu


def _add_body(x_ref, w_ref, o_ref):
    o_ref[...] = x_ref[...] + w_ref[...][None, :, :]


def kernel(x, pos_embed_weight):
    batch, seq, d = x.shape
    s_blk = min(2048, seq)
    n_seq = seq // s_blk
    grid = (n_seq, batch)
    out = pl.pallas_call(
        _add_body,
        grid=grid,
        in_specs=[
            pl.BlockSpec((1, s_blk, d), lambda i, j: (j, i, 0)),
            pl.BlockSpec((s_blk, d), lambda i, j: (i, 0)),
        ],
        out_specs=pl.BlockSpec((1, s_blk, d), lambda i, j: (j, i, 0)),
        out_shape=jax.ShapeDtypeStruct((batch, seq, d), x.dtype),
        compiler_params=pltpu.CompilerParams(
            dimension_semantics=("parallel", "arbitrary"),
        ),
    )(x, pos_embed_weight[:seq])
    return out
